# baseline (speedup 1.0000x reference)
import functools

import jax
import jax.numpy as jnp
from jax import lax
from jax.experimental import pallas as pl
from jax.experimental.pallas import tpu as pltpu
from jax.experimental.pallas import tpu_sc as plsc

B, A, S, H = 2, 32, 1024, 768
VOCAB = 26
VPAD = 32
BA = B * A
NW = 32
SW = S // NW
HV = H // 16


def _stats_kernel(w_ref, p_ref, mean_ref, rstd_ref):
    w = w_ref[...]
    p = p_ref[...]
    mw = jnp.mean(w, axis=1, keepdims=True).reshape(1, VPAD)
    sw = jnp.sum(w * w, axis=1, keepdims=True).reshape(1, VPAD)
    mp = jnp.mean(p, axis=1, keepdims=True)
    sp = jnp.sum(p * p, axis=1, keepdims=True)
    cross = lax.dot_general(p, w, (((1,), (1,)), ((), ())),
                            preferred_element_type=jnp.float32)
    mean = mp + mw
    e2 = (sp + sw + 2.0 * cross) * (1.0 / H)
    var = e2 - mean * mean
    mean_ref[...] = mean
    rstd_ref[...] = lax.rsqrt(var + 1e-12)


def _ln_stats(wpad, pos_s):
    return pl.pallas_call(
        _stats_kernel,
        out_shape=(jax.ShapeDtypeStruct((S, VPAD), jnp.float32),
                   jax.ShapeDtypeStruct((S, VPAD), jnp.float32)),
    )(wpad, pos_s)


def _sc_body(ids_hbm, w_hbm, p_hbm, g_hbm, b_hbm, mean_hbm, rstd_hbm,
             out_hbm, wtab, ptab, gv, bv, mtab, rtab, idsv, obuf,
             sem0, sem1):
    wid = lax.axis_index("s") * 2 + lax.axis_index("c")
    sbase = wid * SW

    pltpu.sync_copy(ids_hbm.at[pl.ds(sbase * BA, SW * BA)], idsv)
    pltpu.sync_copy(w_hbm, wtab)
    pltpu.sync_copy(p_hbm.at[pl.ds(sbase, SW)], ptab)
    pltpu.sync_copy(g_hbm, gv)
    pltpu.sync_copy(b_hbm, bv)
    pltpu.sync_copy(mean_hbm.at[pl.ds(sbase * VPAD, SW * VPAD)], mtab)
    pltpu.sync_copy(rstd_hbm.at[pl.ds(sbase * VPAD, SW * VPAD)], rtab)

    sems = (sem0, sem1)
    iota16 = lax.broadcasted_iota(jnp.int32, (16,), 0)

    def chunk(ba, slot, sem):
        @pl.when(ba >= 2)
        def _():
            pltpu.make_async_copy(
                obuf.at[slot], out_hbm.at[pl.ds(ba * S + sbase, SW)], sem
            ).wait()

        for g16 in range(SW // 16):
            rows16 = iota16 + (g16 * 16)
            idv = plsc.load_gather(idsv, [rows16 * BA + ba])
            meanv = plsc.load_gather(mtab, [rows16 * VPAD + idv])
            rstdv = plsc.load_gather(rtab, [rows16 * VPAD + idv])
            mskv = jnp.where(idv == 0, 0.0, 1.0).astype(jnp.float32)
            av = rstdv * mskv
            cv = -(meanv * av)

            def jloop(j, _, g16=g16, idv=idv, av=av, cv=cv, mskv=mskv):
                jsl = pl.ds(pl.multiple_of(j * 16, 16), 16)
                g = gv[jsl]
                b = bv[jsl]
                for r16 in range(16):
                    r = g16 * 16 + r16
                    tid = idv[r16]
                    v = wtab[tid, jsl] + ptab[r, jsl]
                    o = ((v * av[r16] + cv[r16]) * g + b) * mskv[r16]
                    obuf[slot, r, jsl] = o
                return _

            lax.fori_loop(0, HV, jloop, None)

        pltpu.async_copy(
            obuf.at[slot], out_hbm.at[pl.ds(ba * S + sbase, SW)], sem
        )

    def step(g, _):
        chunk(2 * g, 0, sems[0])
        chunk(2 * g + 1, 1, sems[1])
        return _

    lax.fori_loop(0, BA // 2, step, None)

    pltpu.make_async_copy(
        obuf.at[0], out_hbm.at[pl.ds((BA - 2) * S + sbase, SW)], sems[0]
    ).wait()
    pltpu.make_async_copy(
        obuf.at[1], out_hbm.at[pl.ds((BA - 1) * S + sbase, SW)], sems[1]
    ).wait()


@functools.partial(
    pl.kernel,
    mesh=plsc.VectorSubcoreMesh(core_axis_name="c", subcore_axis_name="s"),
    out_type=jax.ShapeDtypeStruct((BA * S, H), jnp.float32),
    scratch_types=[
        pltpu.VMEM((VPAD, H), jnp.float32),
        pltpu.VMEM((SW, H), jnp.float32),
        pltpu.VMEM((H,), jnp.float32),
        pltpu.VMEM((H,), jnp.float32),
        pltpu.VMEM((SW * VPAD,), jnp.float32),
        pltpu.VMEM((SW * VPAD,), jnp.float32),
        pltpu.VMEM((SW * BA,), jnp.int32),
        pltpu.VMEM((2, SW, H), jnp.float32),
        pltpu.SemaphoreType.DMA,
        pltpu.SemaphoreType.DMA,
    ],
    compiler_params=pltpu.CompilerParams(needs_layout_passes=False),
)
def _sc_embed(ids_hbm, w_hbm, p_hbm, g_hbm, b_hbm, mean_hbm, rstd_hbm,
              out_hbm, *scratch):
    _sc_body(ids_hbm, w_hbm, p_hbm, g_hbm, b_hbm, mean_hbm, rstd_hbm,
             out_hbm, *scratch)


@jax.jit
def kernel(input_ids, word_emb, pos_emb, msa_emb, ln_gamma, ln_beta):
    del msa_emb
    ids2 = input_ids.reshape(BA, S).T.reshape(-1)
    wpad = jnp.zeros((VPAD, H), jnp.float32).at[:VOCAB].set(word_emb)
    pos_s = lax.slice_in_dim(pos_emb, 2, 2 + S, axis=0)
    mean_tab, rstd_tab = _ln_stats(wpad, pos_s)
    out = _sc_embed(ids2, wpad, pos_s, ln_gamma, ln_beta,
                    mean_tab.reshape(-1), rstd_tab.reshape(-1))
    return out.reshape(B, A, S, H)


# speedup vs baseline: 4.7255x; 4.7255x over previous
import functools

import jax
import jax.numpy as jnp
from jax import lax
from jax.experimental import pallas as pl
from jax.experimental.pallas import tpu as pltpu
from jax.experimental.pallas import tpu_sc as plsc

B, A, S, H = 2, 32, 1024, 768
VOCAB = 26
VPAD = 32
BA = B * A
NW = 32
SW = S // NW
HV = H // 16


def _stats_kernel(w_ref, p_ref, g_ref, wc_ref, pc_ref, rstd_ref):
    w = w_ref[...]
    p = p_ref[...]
    g = g_ref[...]
    mw2 = jnp.mean(w, axis=1, keepdims=True)
    mw = mw2.reshape(1, VPAD)
    sw = jnp.sum(w * w, axis=1, keepdims=True).reshape(1, VPAD)
    mp = jnp.mean(p, axis=1, keepdims=True)
    sp = jnp.sum(p * p, axis=1, keepdims=True)
    cross = lax.dot_general(p, w, (((1,), (1,)), ((), ())),
                            preferred_element_type=jnp.float32)
    mean = mp + mw
    e2 = (sp + sw + 2.0 * cross) * (1.0 / H)
    var = e2 - mean * mean
    wc_ref[...] = (w - mw2) * g
    pc_ref[...] = (p - mp) * g
    rstd_ref[...] = lax.rsqrt(var + 1e-12)


def _ln_stats(wpad, pos_s, g2):
    return pl.pallas_call(
        _stats_kernel,
        out_shape=(jax.ShapeDtypeStruct((VPAD, H), jnp.float32),
                   jax.ShapeDtypeStruct((S, H), jnp.float32),
                   jax.ShapeDtypeStruct((S, VPAD), jnp.float32)),
    )(wpad, pos_s, g2)


def _sc_body(ids_hbm, w_hbm, p_hbm, b_hbm, rstd_hbm,
             out_hbm, wtab, ptab, bv, rtab, idsv, obuf,
             sem0, sem1):
    wid = lax.axis_index("s") * 2 + lax.axis_index("c")
    sbase = wid * SW

    pltpu.sync_copy(ids_hbm.at[pl.ds(sbase * BA, SW * BA)], idsv)
    pltpu.sync_copy(w_hbm, wtab)
    pltpu.sync_copy(p_hbm.at[pl.ds(sbase, SW)], ptab)
    pltpu.sync_copy(b_hbm, bv)
    pltpu.sync_copy(rstd_hbm.at[pl.ds(sbase * VPAD, SW * VPAD)], rtab)

    sems = (sem0, sem1)
    iota16 = lax.broadcasted_iota(jnp.int32, (16,), 0)

    def chunk(ba, slot, sem):
        @pl.when(ba >= 2)
        def _():
            pltpu.make_async_copy(
                obuf.at[slot], out_hbm.at[pl.ds(ba * S + sbase, SW)], sem
            ).wait()

        for g16 in range(SW // 16):
            rows16 = iota16 + (g16 * 16)
            idv = plsc.load_gather(idsv, [rows16 * BA + ba])
            rstdv = plsc.load_gather(rtab, [rows16 * VPAD + idv])
            mskv = jnp.where(idv == 0, 0.0, 1.0).astype(jnp.float32)
            av = rstdv * mskv

            for h8 in range(2):
                def jloop(j, g16=g16, h8=h8, idv=idv, av=av, mskv=mskv):
                    jsl = pl.ds(pl.multiple_of(j * 16, 16), 16)
                    b = bv[jsl]
                    for r8 in range(8):
                        r16 = h8 * 8 + r8
                        r = g16 * 16 + r16
                        tid = idv[r16]
                        t = wtab[tid, jsl] + ptab[r, jsl]
                        o = t * av[r16] + b * mskv[r16]
                        obuf[slot, r, jsl] = o

                plsc.parallel_loop(0, HV, 1, unroll=4)(jloop)

        pltpu.async_copy(
            obuf.at[slot], out_hbm.at[pl.ds(ba * S + sbase, SW)], sem
        )

    def step(g, _):
        chunk(2 * g, 0, sems[0])
        chunk(2 * g + 1, 1, sems[1])
        return _

    lax.fori_loop(0, BA // 2, step, None)

    pltpu.make_async_copy(
        obuf.at[0], out_hbm.at[pl.ds((BA - 2) * S + sbase, SW)], sems[0]
    ).wait()
    pltpu.make_async_copy(
        obuf.at[1], out_hbm.at[pl.ds((BA - 1) * S + sbase, SW)], sems[1]
    ).wait()


@functools.partial(
    pl.kernel,
    mesh=plsc.VectorSubcoreMesh(core_axis_name="c", subcore_axis_name="s"),
    out_type=jax.ShapeDtypeStruct((BA * S, H), jnp.float32),
    scratch_types=[
        pltpu.VMEM((VPAD, H), jnp.float32),
        pltpu.VMEM((SW, H), jnp.float32),
        pltpu.VMEM((H,), jnp.float32),
        pltpu.VMEM((SW * VPAD,), jnp.float32),
        pltpu.VMEM((SW * BA,), jnp.int32),
        pltpu.VMEM((2, SW, H), jnp.float32),
        pltpu.SemaphoreType.DMA,
        pltpu.SemaphoreType.DMA,
    ],
    compiler_params=pltpu.CompilerParams(needs_layout_passes=False),
)
def _sc_embed(ids_hbm, w_hbm, p_hbm, b_hbm, rstd_hbm, out_hbm, *scratch):
    _sc_body(ids_hbm, w_hbm, p_hbm, b_hbm, rstd_hbm, out_hbm, *scratch)


@jax.jit
def kernel(input_ids, word_emb, pos_emb, msa_emb, ln_gamma, ln_beta):
    del msa_emb
    ids2 = input_ids.reshape(BA, S).T.reshape(-1)
    wpad = jnp.zeros((VPAD, H), jnp.float32).at[:VOCAB].set(word_emb)
    pos_s = lax.slice_in_dim(pos_emb, 2, 2 + S, axis=0)
    wc, pc, rstd_tab = _ln_stats(wpad, pos_s, ln_gamma.reshape(1, H))

    out = _sc_embed(ids2, wc, pc, ln_beta, rstd_tab.reshape(-1))
    return out.reshape(B, A, S, H)


# speedup vs baseline: 4.9313x; 1.0435x over previous
import functools

import jax
import jax.numpy as jnp
from jax import lax
from jax.experimental import pallas as pl
from jax.experimental.pallas import tpu as pltpu
from jax.experimental.pallas import tpu_sc as plsc

B, A, S, H = 2, 32, 1024, 768
VOCAB = 26
VPAD = 32
BA = B * A
NW = 32
SW = S // NW
HV = H // 16


def _stats_kernel(w_ref, p_ref, g_ref, wc_ref, pc_ref, rstd_ref):
    w = w_ref[...]
    p = p_ref[...]
    g = g_ref[...]
    mw2 = jnp.mean(w, axis=1, keepdims=True)
    mw = mw2.reshape(1, VPAD)
    sw = jnp.sum(w * w, axis=1, keepdims=True).reshape(1, VPAD)
    mp = jnp.mean(p, axis=1, keepdims=True)
    sp = jnp.sum(p * p, axis=1, keepdims=True)
    cross = lax.dot_general(p, w, (((1,), (1,)), ((), ())),
                            preferred_element_type=jnp.float32)
    mean = mp + mw
    e2 = (sp + sw + 2.0 * cross) * (1.0 / H)
    var = e2 - mean * mean
    wc_ref[...] = (w - mw2) * g
    pc_ref[...] = (p - mp) * g
    rstd_ref[...] = lax.rsqrt(var + 1e-12)


def _ln_stats(wpad, pos_s, g2):
    return pl.pallas_call(
        _stats_kernel,
        out_shape=(jax.ShapeDtypeStruct((VPAD, H), jnp.float32),
                   jax.ShapeDtypeStruct((S, H), jnp.float32),
                   jax.ShapeDtypeStruct((S, VPAD), jnp.float32)),
    )(wpad, pos_s, g2)


def _sc_body(ids_hbm, w_hbm, p_hbm, b_hbm, rstd_hbm,
             out_hbm, wtab, ptab, bv, rtab, idsv, obuf,
             sem0, sem1):
    wid = lax.axis_index("s") * 2 + lax.axis_index("c")
    sbase = wid * SW

    pltpu.async_copy(ids_hbm.at[pl.ds(sbase * BA, SW * BA)], idsv, sem0)
    pltpu.async_copy(w_hbm, wtab, sem0)
    pltpu.async_copy(p_hbm.at[pl.ds(sbase, SW)], ptab, sem0)
    pltpu.async_copy(b_hbm, bv, sem0)
    pltpu.async_copy(rstd_hbm.at[pl.ds(sbase * VPAD, SW * VPAD)], rtab, sem0)
    pltpu.make_async_copy(ids_hbm.at[pl.ds(sbase * BA, SW * BA)], idsv,
                          sem0).wait()
    pltpu.make_async_copy(w_hbm, wtab, sem0).wait()
    pltpu.make_async_copy(p_hbm.at[pl.ds(sbase, SW)], ptab, sem0).wait()
    pltpu.make_async_copy(b_hbm, bv, sem0).wait()
    pltpu.make_async_copy(rstd_hbm.at[pl.ds(sbase * VPAD, SW * VPAD)], rtab,
                          sem0).wait()

    sems = (sem0, sem1)
    iota16 = lax.broadcasted_iota(jnp.int32, (16,), 0)

    def chunk(ba, slot, sem):
        @pl.when(ba >= 2)
        def _():
            pltpu.make_async_copy(
                obuf.at[slot], out_hbm.at[pl.ds(ba * S + sbase, SW)], sem
            ).wait()

        for g16 in range(SW // 16):
            rows16 = iota16 + (g16 * 16)
            idv = plsc.load_gather(idsv, [rows16 * BA + ba])
            rstdv = plsc.load_gather(rtab, [rows16 * VPAD + idv])
            mskv = jnp.where(idv == 0, 0.0, 1.0).astype(jnp.float32)
            av = rstdv * mskv

            for h8 in range(2):
                def jloop(j, g16=g16, h8=h8, idv=idv, av=av, mskv=mskv):
                    jsl = pl.ds(pl.multiple_of(j * 16, 16), 16)
                    b = bv[jsl]
                    for r8 in range(8):
                        r16 = h8 * 8 + r8
                        r = g16 * 16 + r16
                        tid = idv[r16]
                        t = wtab[tid, jsl] + ptab[r, jsl]
                        o = t * av[r16] + b * mskv[r16]
                        obuf[slot, r, jsl] = o

                plsc.parallel_loop(0, HV, 1, unroll=2)(jloop)

        pltpu.async_copy(
            obuf.at[slot], out_hbm.at[pl.ds(ba * S + sbase, SW)], sem
        )

    def step(g, _):
        chunk(2 * g, 0, sems[0])
        chunk(2 * g + 1, 1, sems[1])
        return _

    lax.fori_loop(0, BA // 2, step, None)

    pltpu.make_async_copy(
        obuf.at[0], out_hbm.at[pl.ds((BA - 2) * S + sbase, SW)], sems[0]
    ).wait()
    pltpu.make_async_copy(
        obuf.at[1], out_hbm.at[pl.ds((BA - 1) * S + sbase, SW)], sems[1]
    ).wait()


@functools.partial(
    pl.kernel,
    mesh=plsc.VectorSubcoreMesh(core_axis_name="c", subcore_axis_name="s"),
    out_type=jax.ShapeDtypeStruct((BA * S, H), jnp.float32),
    scratch_types=[
        pltpu.VMEM((VPAD, H), jnp.float32),
        pltpu.VMEM((SW, H), jnp.float32),
        pltpu.VMEM((H,), jnp.float32),
        pltpu.VMEM((SW * VPAD,), jnp.float32),
        pltpu.VMEM((SW * BA,), jnp.int32),
        pltpu.VMEM((2, SW, H), jnp.float32),
        pltpu.SemaphoreType.DMA,
        pltpu.SemaphoreType.DMA,
    ],
    compiler_params=pltpu.CompilerParams(needs_layout_passes=False),
)
def _sc_embed(ids_hbm, w_hbm, p_hbm, b_hbm, rstd_hbm, out_hbm, *scratch):
    _sc_body(ids_hbm, w_hbm, p_hbm, b_hbm, rstd_hbm, out_hbm, *scratch)


@jax.jit
def kernel(input_ids, word_emb, pos_emb, msa_emb, ln_gamma, ln_beta):
    del msa_emb
    ids2 = input_ids.reshape(BA, S).T.reshape(-1)
    wpad = jnp.zeros((VPAD, H), jnp.float32).at[:VOCAB].set(word_emb)
    pos_s = lax.slice_in_dim(pos_emb, 2, 2 + S, axis=0)
    wc, pc, rstd_tab = _ln_stats(wpad, pos_s, ln_gamma.reshape(1, H))

    out = _sc_embed(ids2, wc, pc, ln_beta, rstd_tab.reshape(-1))
    return out.reshape(B, A, S, H)
